# baseline (speedup 1.0000x reference)
import functools

import jax
import jax.numpy as jnp
from jax import lax
from jax.experimental import pallas as pl
from jax.experimental.pallas import tpu as pltpu
from jax.experimental.pallas import tpu_sc as plsc


def _sc_gather(table, idx, n_rows, d_pad):
    dtype = table.dtype
    itemsize = jnp.dtype(dtype).itemsize
    info = plsc.get_sparse_core_info()
    nc, ns = info.num_cores, info.num_subcores
    nw = nc * ns
    rows_pw = n_rows // nw
    chunk = rows_pw
    while chunk * d_pad * itemsize * 2 + rows_pw * 4 > 440_000:
        chunk //= 2
    n_chunks = rows_pw // chunk
    mesh = plsc.VectorSubcoreMesh(core_axis_name="c", subcore_axis_name="s")

    @functools.partial(
        pl.kernel,
        out_type=jax.ShapeDtypeStruct((n_rows, d_pad), dtype),
        mesh=mesh,
        scratch_types=[
            pltpu.VMEM((rows_pw,), jnp.int32),
            pltpu.VMEM((chunk, d_pad), dtype),
            pltpu.VMEM((chunk, d_pad), dtype),
            pltpu.SemaphoreType.DMA,
            pltpu.SemaphoreType.DMA,
        ],
    )
    def k(table_hbm, idx_hbm, out_hbm, idx_v, rows_a, rows_b, sem_a, sem_b):
        wid = lax.axis_index("s") * nc + lax.axis_index("c")
        base = wid * rows_pw
        bufs = (rows_a, rows_b)
        sems = (sem_a, sem_b)
        pltpu.sync_copy(idx_hbm.at[pl.ds(base, rows_pw)], idx_v)
        pltpu.async_copy(
            table_hbm.at[idx_v.at[pl.ds(0, chunk)]], bufs[0], sems[0])
        for j in range(n_chunks):
            if j + 1 < n_chunks:
                pltpu.async_copy(
                    table_hbm.at[idx_v.at[pl.ds((j + 1) * chunk, chunk)]],
                    bufs[(j + 1) % 2], sems[(j + 1) % 2])
            pltpu.make_async_copy(
                table_hbm.at[idx_v.at[pl.ds(j * chunk, chunk)]],
                bufs[j % 2], sems[j % 2]).wait()
            pltpu.sync_copy(bufs[j % 2], out_hbm.at[pl.ds(base + j * chunk, chunk)])

    return k(table, idx)


def _pad_rows(emb, d_pad):
    V, D = emb.shape
    blk = 4000
    grid = (V // blk,) if V % blk == 0 else ((V + blk - 1) // blk,)

    def body(x_ref, o_ref):
        padded = jnp.pad(x_ref[...], ((0, 0), (0, d_pad - D)))
        col = lax.broadcasted_iota(jnp.int32, padded.shape, 1)
        o_ref[...] = jnp.where(col == D, 1.0, padded)

    return pl.pallas_call(
        body,
        grid=grid,
        in_specs=[pl.BlockSpec((blk, D), lambda i: (i, 0))],
        out_specs=pl.BlockSpec((blk, d_pad), lambda i: (i, 0)),
        out_shape=jax.ShapeDtypeStruct((grid[0] * blk, d_pad), emb.dtype),
    )(emb)[:V]


def _bilstm_step(
    xf_ref, xb_ref,
    wif_ref, wib_ref,
    wof_ref, wob_ref, bo_ref,
    outf_ref, outb_ref,
    hf, cf, hb, cb,
    *, hidden, tblk,
):
    t = pl.program_id(0)

    @pl.when(t == 0)
    def _init():
        hf[...] = jnp.zeros_like(hf)
        cf[...] = jnp.zeros_like(cf)
        hb[...] = jnp.zeros_like(hb)
        cb[...] = jnp.zeros_like(cb)

    h = hidden

    def step(x, w_cat, h2_prev, c_prev):
        xh = jnp.concatenate([x.astype(jnp.bfloat16), h2_prev], axis=1)
        g = jnp.dot(xh, w_cat, preferred_element_type=jnp.float32)
        ti = jnp.tanh(g[:, 0 * h:1 * h])
        tf = jnp.tanh(g[:, 1 * h:2 * h])
        gg = jnp.tanh(g[:, 2 * h:3 * h])
        to = jnp.tanh(g[:, 3 * h:4 * h])
        c = 0.5 * ((tf + 1.0) * c_prev + (ti + 1.0) * gg)
        return ((to + 1.0) * jnp.tanh(c)).astype(jnp.bfloat16), c

    h_f, c_f = hf[...], cf[...]
    h_b, c_b = hb[...], cb[...]
    for k in range(tblk):
        h_f, c_f = step(xf_ref[k], wif_ref[...], h_f, c_f)
        outf_ref[k] = (
            jnp.dot(h_f, wof_ref[...], preferred_element_type=jnp.float32)
            + bo_ref[...]
        )
        h_b, c_b = step(xb_ref[tblk - 1 - k], wib_ref[...], h_b, c_b)
        outb_ref[tblk - 1 - k] = jnp.dot(
            h_b, wob_ref[...], preferred_element_type=jnp.float32)
    hf[...] = h_f
    cf[...] = c_f
    hb[...] = h_b
    cb[...] = c_b


def _bilstm(xs, wif, wib, wof, wob, bo):
    L, B, Dp = xs.shape
    H = wof.shape[0]
    O = wof.shape[1]
    f32 = jnp.float32
    T = next((t for t in (4, 5, 2, 1) if L % t == 0), 1)
    LT = L // T
    grid = (LT,)
    const = lambda t: (0, 0)
    out_f, out_b = pl.pallas_call(
        functools.partial(_bilstm_step, hidden=H, tblk=T),
        grid=grid,
        in_specs=[
            pl.BlockSpec((T, B, Dp), lambda t: (t, 0, 0)),
            pl.BlockSpec((T, B, Dp), lambda t: (LT - 1 - t, 0, 0)),
            pl.BlockSpec((Dp + H, 4 * H), const),
            pl.BlockSpec((Dp + H, 4 * H), const),
            pl.BlockSpec((H, O), const),
            pl.BlockSpec((H, O), const),
            pl.BlockSpec((1, O), const),
        ],
        out_specs=[
            pl.BlockSpec((T, B, O), lambda t: (t, 0, 0)),
            pl.BlockSpec((T, B, O), lambda t: (LT - 1 - t, 0, 0)),
        ],
        out_shape=[
            jax.ShapeDtypeStruct((L, B, O), f32),
            jax.ShapeDtypeStruct((L, B, O), f32),
        ],
        scratch_shapes=[
            pltpu.VMEM((B, H), jnp.bfloat16), pltpu.VMEM((B, H), f32),
            pltpu.VMEM((B, H), jnp.bfloat16), pltpu.VMEM((B, H), f32),
        ],
        compiler_params=pltpu.CompilerParams(
            dimension_semantics=("arbitrary",),
        ),
    )(xs, xs, wif, wib, wof, wob, bo)
    return out_f, out_b


def kernel(sentence, emb, W_ih_f, W_hh_f, b_ih_f, b_hh_f,
           W_ih_b, W_hh_b, b_ih_b, b_hh_b, W_out, b_out):
    B, L = sentence.shape
    V, D = emb.shape
    H = W_hh_f.shape[1]
    O = W_out.shape[0]
    Dp = (D + 127) // 128 * 128
    bf16 = jnp.bfloat16

    emb_p = _pad_rows(emb, Dp)
    idx = sentence.astype(jnp.int32).T.reshape(-1)

    xs = _sc_gather(emb_p, idx, B * L, Dp).reshape(L, B, Dp)

    gscale = jnp.concatenate([
        jnp.full((1, H), 0.5), jnp.full((1, H), 0.5),
        jnp.ones((1, H)), jnp.full((1, H), 0.5)], axis=1).astype(jnp.float32)

    def wcat(W_ih, W_hh, b_ih, b_hh):
        wx = jnp.pad(W_ih.T, ((0, Dp - D), (0, 0)))
        wx = wx.at[D].set(b_ih + b_hh)
        w = jnp.concatenate([wx, 0.5 * W_hh.T], axis=0)
        return (w * gscale).astype(bf16)

    wif = wcat(W_ih_f, W_hh_f, b_ih_f, b_hh_f)
    wib = wcat(W_ih_b, W_hh_b, b_ih_b, b_hh_b)
    wof = (0.5 * W_out[:, :H].T).astype(bf16)
    wob = (0.5 * W_out[:, H:].T).astype(bf16)
    bo = b_out.reshape(1, O)

    out_f, out_b = _bilstm(xs, wif, wib, wof, wob, bo)
    return jnp.swapaxes(out_f + out_b, 0, 1)


# speedup vs baseline: 1.0154x; 1.0154x over previous
import functools

import jax
import jax.numpy as jnp
from jax import lax
from jax.experimental import pallas as pl
from jax.experimental.pallas import tpu as pltpu
from jax.experimental.pallas import tpu_sc as plsc


def _sc_gather(table, idx, n_rows, d_pad):
    dtype = table.dtype
    itemsize = jnp.dtype(dtype).itemsize
    info = plsc.get_sparse_core_info()
    nc, ns = info.num_cores, info.num_subcores
    nw = nc * ns
    rows_pw = n_rows // nw
    chunk = rows_pw
    while chunk * d_pad * itemsize * 2 + rows_pw * 4 > 440_000:
        chunk //= 2
    n_chunks = rows_pw // chunk
    mesh = plsc.VectorSubcoreMesh(core_axis_name="c", subcore_axis_name="s")

    @functools.partial(
        pl.kernel,
        out_type=jax.ShapeDtypeStruct((n_rows, d_pad), dtype),
        mesh=mesh,
        scratch_types=[
            pltpu.VMEM((rows_pw,), jnp.int32),
            pltpu.VMEM((chunk, d_pad), dtype),
            pltpu.VMEM((chunk, d_pad), dtype),
            pltpu.SemaphoreType.DMA,
            pltpu.SemaphoreType.DMA,
            pltpu.SemaphoreType.DMA,
            pltpu.SemaphoreType.DMA,
        ],
    )
    def k(table_hbm, idx_hbm, out_hbm, idx_v, rows_a, rows_b,
          gsem_a, gsem_b, wsem_a, wsem_b):
        wid = lax.axis_index("s") * nc + lax.axis_index("c")
        base = wid * rows_pw
        bufs = (rows_a, rows_b)
        gsems = (gsem_a, gsem_b)
        wsems = (wsem_a, wsem_b)
        pltpu.sync_copy(idx_hbm.at[pl.ds(base, rows_pw)], idx_v)
        pltpu.async_copy(
            table_hbm.at[idx_v.at[pl.ds(0, chunk)]], bufs[0], gsems[0])
        for j in range(n_chunks):
            p = j % 2
            q = (j + 1) % 2
            if j + 1 < n_chunks:
                if j >= 1:
                    pltpu.make_async_copy(
                        bufs[q],
                        out_hbm.at[pl.ds(base + (j - 1) * chunk, chunk)],
                        wsems[q]).wait()
                pltpu.async_copy(
                    table_hbm.at[idx_v.at[pl.ds((j + 1) * chunk, chunk)]],
                    bufs[q], gsems[q])
            pltpu.make_async_copy(
                table_hbm.at[idx_v.at[pl.ds(j * chunk, chunk)]],
                bufs[p], gsems[p]).wait()
            pltpu.async_copy(
                bufs[p], out_hbm.at[pl.ds(base + j * chunk, chunk)], wsems[p])
        for j in (n_chunks - 2, n_chunks - 1):
            pltpu.make_async_copy(
                bufs[j % 2], out_hbm.at[pl.ds(base + j * chunk, chunk)],
                wsems[j % 2]).wait()

    return k(table, idx)


def _pad_rows(emb, d_pad):
    V, D = emb.shape
    blk = 4000
    grid = (V // blk,) if V % blk == 0 else ((V + blk - 1) // blk,)

    def body(x_ref, o_ref):
        padded = jnp.pad(x_ref[...], ((0, 0), (0, d_pad - D)))
        col = lax.broadcasted_iota(jnp.int32, padded.shape, 1)
        o_ref[...] = jnp.where(col == D, 1.0, padded)

    return pl.pallas_call(
        body,
        grid=grid,
        in_specs=[pl.BlockSpec((blk, D), lambda i: (i, 0))],
        out_specs=pl.BlockSpec((blk, d_pad), lambda i: (i, 0)),
        out_shape=jax.ShapeDtypeStruct((grid[0] * blk, d_pad), emb.dtype),
    )(emb)[:V]


def _bilstm_step(
    xf_ref, xb_ref,
    wif_ref, wib_ref,
    wof_ref, wob_ref, bo_ref,
    outf_ref, outb_ref,
    hf, cf, hb, cb,
    *, hidden, tblk,
):
    t = pl.program_id(0)

    @pl.when(t == 0)
    def _init():
        hf[...] = jnp.zeros_like(hf)
        cf[...] = jnp.zeros_like(cf)
        hb[...] = jnp.zeros_like(hb)
        cb[...] = jnp.zeros_like(cb)

    h = hidden

    def step(x, w_cat, h2_prev, c_prev):
        xh = jnp.concatenate([x.astype(jnp.bfloat16), h2_prev], axis=1)
        g = jnp.dot(xh, w_cat, preferred_element_type=jnp.float32)
        ti = jnp.tanh(g[:, 0 * h:1 * h])
        tf = jnp.tanh(g[:, 1 * h:2 * h])
        gg = jnp.tanh(g[:, 2 * h:3 * h])
        to = jnp.tanh(g[:, 3 * h:4 * h])
        c = 0.5 * ((tf + 1.0) * c_prev + (ti + 1.0) * gg)
        return ((to + 1.0) * jnp.tanh(c)).astype(jnp.bfloat16), c

    h_f, c_f = hf[...], cf[...]
    h_b, c_b = hb[...], cb[...]
    for k in range(tblk):
        h_f, c_f = step(xf_ref[k], wif_ref[...], h_f, c_f)
        outf_ref[k] = (
            jnp.dot(h_f, wof_ref[...], preferred_element_type=jnp.float32)
            + bo_ref[...]
        )
        h_b, c_b = step(xb_ref[tblk - 1 - k], wib_ref[...], h_b, c_b)
        outb_ref[tblk - 1 - k] = jnp.dot(
            h_b, wob_ref[...], preferred_element_type=jnp.float32)
    hf[...] = h_f
    cf[...] = c_f
    hb[...] = h_b
    cb[...] = c_b


def _bilstm(xs, wif, wib, wof, wob, bo):
    L, B, Dp = xs.shape
    H = wof.shape[0]
    O = wof.shape[1]
    f32 = jnp.float32
    T = next((t for t in (8, 4, 5, 2, 1) if L % t == 0), 1)
    LT = L // T
    grid = (LT,)
    const = lambda t: (0, 0)
    out_f, out_b = pl.pallas_call(
        functools.partial(_bilstm_step, hidden=H, tblk=T),
        grid=grid,
        in_specs=[
            pl.BlockSpec((T, B, Dp), lambda t: (t, 0, 0)),
            pl.BlockSpec((T, B, Dp), lambda t: (LT - 1 - t, 0, 0)),
            pl.BlockSpec((Dp + H, 4 * H), const),
            pl.BlockSpec((Dp + H, 4 * H), const),
            pl.BlockSpec((H, O), const),
            pl.BlockSpec((H, O), const),
            pl.BlockSpec((1, O), const),
        ],
        out_specs=[
            pl.BlockSpec((T, B, O), lambda t: (t, 0, 0)),
            pl.BlockSpec((T, B, O), lambda t: (LT - 1 - t, 0, 0)),
        ],
        out_shape=[
            jax.ShapeDtypeStruct((L, B, O), f32),
            jax.ShapeDtypeStruct((L, B, O), f32),
        ],
        scratch_shapes=[
            pltpu.VMEM((B, H), jnp.bfloat16), pltpu.VMEM((B, H), f32),
            pltpu.VMEM((B, H), jnp.bfloat16), pltpu.VMEM((B, H), f32),
        ],
        compiler_params=pltpu.CompilerParams(
            dimension_semantics=("arbitrary",),
        ),
    )(xs, xs, wif, wib, wof, wob, bo)
    return out_f, out_b


def kernel(sentence, emb, W_ih_f, W_hh_f, b_ih_f, b_hh_f,
           W_ih_b, W_hh_b, b_ih_b, b_hh_b, W_out, b_out):
    B, L = sentence.shape
    V, D = emb.shape
    H = W_hh_f.shape[1]
    O = W_out.shape[0]
    Dp = (D + 127) // 128 * 128
    bf16 = jnp.bfloat16

    emb_p = _pad_rows(emb, Dp)
    idx = sentence.astype(jnp.int32).T.reshape(-1)

    xs = _sc_gather(emb_p, idx, B * L, Dp).reshape(L, B, Dp)

    gscale = jnp.concatenate([
        jnp.full((1, H), 0.5), jnp.full((1, H), 0.5),
        jnp.ones((1, H)), jnp.full((1, H), 0.5)], axis=1).astype(jnp.float32)

    def wcat(W_ih, W_hh, b_ih, b_hh):
        wx = jnp.pad(W_ih.T, ((0, Dp - D), (0, 0)))
        wx = wx.at[D].set(b_ih + b_hh)
        w = jnp.concatenate([wx, 0.5 * W_hh.T], axis=0)
        return (w * gscale).astype(bf16)

    wif = wcat(W_ih_f, W_hh_f, b_ih_f, b_hh_f)
    wib = wcat(W_ih_b, W_hh_b, b_ih_b, b_hh_b)
    wof = (0.5 * W_out[:, :H].T).astype(bf16)
    wob = (0.5 * W_out[:, H:].T).astype(bf16)
    bo = b_out.reshape(1, O)

    out_f, out_b = _bilstm(xs, wif, wib, wof, wob, bo)
    return jnp.swapaxes(out_f + out_b, 0, 1)


# speedup vs baseline: 1.1829x; 1.1650x over previous
import functools

import jax
import jax.numpy as jnp
from jax import lax
from jax.experimental import pallas as pl
from jax.experimental.pallas import tpu as pltpu
from jax.experimental.pallas import tpu_sc as plsc


def _sc_gather(table, idx, n_rows, d_pad):
    dtype = table.dtype
    itemsize = jnp.dtype(dtype).itemsize
    info = plsc.get_sparse_core_info()
    nc, ns = info.num_cores, info.num_subcores
    nw = nc * ns
    rows_pw = n_rows // nw
    chunk = rows_pw
    while chunk * d_pad * itemsize * 2 + rows_pw * 4 > 440_000:
        chunk //= 2
    n_chunks = rows_pw // chunk
    mesh = plsc.VectorSubcoreMesh(core_axis_name="c", subcore_axis_name="s")

    @functools.partial(
        pl.kernel,
        out_type=jax.ShapeDtypeStruct((n_rows, d_pad), dtype),
        mesh=mesh,
        scratch_types=[
            pltpu.VMEM((rows_pw,), jnp.int32),
            pltpu.VMEM((chunk, d_pad), dtype),
            pltpu.VMEM((chunk, d_pad), dtype),
            pltpu.SemaphoreType.DMA,
            pltpu.SemaphoreType.DMA,
            pltpu.SemaphoreType.DMA,
            pltpu.SemaphoreType.DMA,
        ],
    )
    def k(table_hbm, idx_hbm, out_hbm, idx_v, rows_a, rows_b,
          gsem_a, gsem_b, wsem_a, wsem_b):
        wid = lax.axis_index("s") * nc + lax.axis_index("c")
        base = wid * rows_pw
        bufs = (rows_a, rows_b)
        gsems = (gsem_a, gsem_b)
        wsems = (wsem_a, wsem_b)
        pltpu.sync_copy(idx_hbm.at[pl.ds(base, rows_pw)], idx_v)
        pltpu.async_copy(
            table_hbm.at[idx_v.at[pl.ds(0, chunk)]], bufs[0], gsems[0])
        for j in range(n_chunks):
            p = j % 2
            q = (j + 1) % 2
            if j + 1 < n_chunks:
                if j >= 1:
                    pltpu.make_async_copy(
                        bufs[q],
                        out_hbm.at[pl.ds(base + (j - 1) * chunk, chunk)],
                        wsems[q]).wait()
                pltpu.async_copy(
                    table_hbm.at[idx_v.at[pl.ds((j + 1) * chunk, chunk)]],
                    bufs[q], gsems[q])
            pltpu.make_async_copy(
                table_hbm.at[idx_v.at[pl.ds(j * chunk, chunk)]],
                bufs[p], gsems[p]).wait()
            pltpu.async_copy(
                bufs[p], out_hbm.at[pl.ds(base + j * chunk, chunk)], wsems[p])
        for j in (n_chunks - 2, n_chunks - 1):
            pltpu.make_async_copy(
                bufs[j % 2], out_hbm.at[pl.ds(base + j * chunk, chunk)],
                wsems[j % 2]).wait()

    return k(table, idx)


def _pad_rows(emb_t, d_pad):
    D, V = emb_t.shape
    blk = 4096
    grid = ((V + blk - 1) // blk,)

    def body(x_ref, o_ref):
        xt = jnp.swapaxes(x_ref[...], 0, 1)
        padded = jnp.pad(xt, ((0, 0), (0, d_pad - D)))
        col = lax.broadcasted_iota(jnp.int32, padded.shape, 1)
        o_ref[...] = jnp.where(col == D, 1.0, padded)

    return pl.pallas_call(
        body,
        grid=grid,
        in_specs=[pl.BlockSpec((D, blk), lambda i: (0, i))],
        out_specs=pl.BlockSpec((blk, d_pad), lambda i: (i, 0)),
        out_shape=jax.ShapeDtypeStruct((grid[0] * blk, d_pad), emb_t.dtype),
    )(emb_t)


def _bilstm_step(
    xf_ref, xb_ref,
    wif_ref, wib_ref,
    wof_ref, wob_ref, bo_ref,
    outf_ref, outb_ref,
    hf, cf, hb, cb,
    *, hidden, tblk,
):
    t = pl.program_id(0)

    @pl.when(t == 0)
    def _init():
        hf[...] = jnp.zeros_like(hf)
        cf[...] = jnp.zeros_like(cf)
        hb[...] = jnp.zeros_like(hb)
        cb[...] = jnp.zeros_like(cb)

    h = hidden

    def step(x, w_cat, h2_prev, c_prev):
        xh = jnp.concatenate([x.astype(jnp.bfloat16), h2_prev], axis=1)
        g = jnp.dot(xh, w_cat, preferred_element_type=jnp.float32)
        ti = jnp.tanh(g[:, 0 * h:1 * h])
        tf = jnp.tanh(g[:, 1 * h:2 * h])
        gg = jnp.tanh(g[:, 2 * h:3 * h])
        to = jnp.tanh(g[:, 3 * h:4 * h])
        c = 0.5 * ((tf + 1.0) * c_prev + (ti + 1.0) * gg)
        return ((to + 1.0) * jnp.tanh(c)).astype(jnp.bfloat16), c

    h_f, c_f = hf[...], cf[...]
    h_b, c_b = hb[...], cb[...]
    for k in range(tblk):
        h_f, c_f = step(xf_ref[k], wif_ref[...], h_f, c_f)
        pf = (jnp.dot(h_f, wof_ref[...], preferred_element_type=jnp.float32)
              + bo_ref[...])
        outf_ref[k] = pf.astype(jnp.bfloat16)
        h_b, c_b = step(xb_ref[tblk - 1 - k], wib_ref[...], h_b, c_b)
        pb = jnp.dot(h_b, wob_ref[...], preferred_element_type=jnp.float32)
        outb_ref[tblk - 1 - k] = pb.astype(jnp.bfloat16)
    hf[...] = h_f
    cf[...] = c_f
    hb[...] = h_b
    cb[...] = c_b


def _bilstm(xs, wif, wib, wof, wob, bo):
    L, B, Dp = xs.shape
    H = wof.shape[0]
    O = wof.shape[1]
    f32 = jnp.float32
    T = next((t for t in (8, 4, 5, 2, 1) if L % t == 0), 1)
    LT = L // T
    grid = (LT,)
    const = lambda t: (0, 0)
    out_f, out_b = pl.pallas_call(
        functools.partial(_bilstm_step, hidden=H, tblk=T),
        grid=grid,
        in_specs=[
            pl.BlockSpec((T, B, Dp), lambda t: (t, 0, 0)),
            pl.BlockSpec((T, B, Dp), lambda t: (LT - 1 - t, 0, 0)),
            pl.BlockSpec((Dp + H, 4 * H), const),
            pl.BlockSpec((Dp + H, 4 * H), const),
            pl.BlockSpec((H, O), const),
            pl.BlockSpec((H, O), const),
            pl.BlockSpec((1, O), const),
        ],
        out_specs=[
            pl.BlockSpec((T, B, O), lambda t: (t, 0, 0)),
            pl.BlockSpec((T, B, O), lambda t: (LT - 1 - t, 0, 0)),
        ],
        out_shape=[
            jax.ShapeDtypeStruct((L, B, O), jnp.bfloat16),
            jax.ShapeDtypeStruct((L, B, O), jnp.bfloat16),
        ],
        scratch_shapes=[
            pltpu.VMEM((B, H), jnp.bfloat16), pltpu.VMEM((B, H), f32),
            pltpu.VMEM((B, H), jnp.bfloat16), pltpu.VMEM((B, H), f32),
        ],
        compiler_params=pltpu.CompilerParams(
            dimension_semantics=("arbitrary",),
        ),
    )(xs, xs, wif, wib, wof, wob, bo)
    return out_f, out_b


def kernel(sentence, emb, W_ih_f, W_hh_f, b_ih_f, b_hh_f,
           W_ih_b, W_hh_b, b_ih_b, b_hh_b, W_out, b_out):
    B, L = sentence.shape
    V, D = emb.shape
    H = W_hh_f.shape[1]
    O = W_out.shape[0]
    Dp = (D + 127) // 128 * 128
    bf16 = jnp.bfloat16

    emb_p = _pad_rows(jnp.swapaxes(emb, 0, 1), Dp)
    idx = sentence.astype(jnp.int32).T.reshape(-1)

    xs = _sc_gather(emb_p, idx, B * L, Dp).reshape(L, B, Dp)

    gscale = jnp.concatenate([
        jnp.full((1, H), 0.5), jnp.full((1, H), 0.5),
        jnp.ones((1, H)), jnp.full((1, H), 0.5)], axis=1).astype(jnp.float32)

    def wcat(W_ih, W_hh, b_ih, b_hh):
        wx = jnp.pad(W_ih.T, ((0, Dp - D), (0, 0)))
        wx = wx.at[D].set(b_ih + b_hh)
        w = jnp.concatenate([wx, 0.5 * W_hh.T], axis=0)
        return (w * gscale).astype(bf16)

    wif = wcat(W_ih_f, W_hh_f, b_ih_f, b_hh_f)
    wib = wcat(W_ih_b, W_hh_b, b_ih_b, b_hh_b)
    wof = (0.5 * W_out[:, :H].T).astype(bf16)
    wob = (0.5 * W_out[:, H:].T).astype(bf16)
    bo = b_out.reshape(1, O)

    out_f, out_b = _bilstm(xs, wif, wib, wof, wob, bo)
    return jnp.swapaxes(out_f.astype(jnp.float32) + out_b.astype(jnp.float32),
                        0, 1)
